# Initial kernel scaffold; baseline (speedup 1.0000x reference)
import jax
import jax.numpy as jnp
from jax.experimental import pallas as pl


def kernel(x, edge_index, W1, b1, W2, b2):
    raise NotImplementedError("write your pallas kernel here")


# speedup vs baseline: 19.7201x; 19.7201x over previous
import functools

import jax
import jax.numpy as jnp
from jax import lax
from jax.experimental import pallas as pl
from jax.experimental.pallas import tpu as pltpu
from jax.experimental.pallas import tpu_sc as plsc

N = 10000
D = 128
NPAD = 10240
NC = 2
NS = 16
NW = NC * NS
C = 128
NCHUNKS = 80
EPW = NCHUNKS * C
EPAD = NW * EPW
ROWS_PT = NPAD // NS
TCB = 1024


def _sc_mesh():
  return plsc.VectorSubcoreMesh(core_axis_name="c", subcore_axis_name="s")


@functools.partial(
    pl.kernel,
    mesh=_sc_mesh(),
    out_type=jax.ShapeDtypeStruct((NC, NPAD, D), jnp.float32),
    scratch_types=[
        pltpu.VMEM_SHARED((NPAD, D), jnp.float32),
        pltpu.VMEM((NCHUNKS, C), jnp.int32),
        pltpu.VMEM((C, D), jnp.float32),
    ],
)
def _deg_sc(dsts_hbm, ones_hbm, zeros_hbm, out_hbm, acc, dst_v, ones_v):
  cid = lax.axis_index("c")
  sid = lax.axis_index("s")
  wid = sid * NC + cid
  base = sid * ROWS_PT
  pltpu.sync_copy(zeros_hbm.at[pl.ds(base, ROWS_PT)],
                  acc.at[pl.ds(base, ROWS_PT)])
  pltpu.sync_copy(dsts_hbm.at[wid], dst_v)
  pltpu.sync_copy(ones_hbm, ones_v)
  plsc.subcore_barrier()

  def body(j, carry):
    pltpu.sync_copy(ones_v, acc.at[dst_v.at[j]], add=True)
    return carry

  lax.fori_loop(0, NCHUNKS, body, None)
  plsc.subcore_barrier()
  pltpu.sync_copy(acc.at[pl.ds(base, ROWS_PT)],
                  out_hbm.at[cid, pl.ds(base, ROWS_PT)])


@functools.partial(
    pl.kernel,
    mesh=_sc_mesh(),
    out_type=jax.ShapeDtypeStruct((NC, NPAD, D), jnp.float32),
    scratch_types=[
        pltpu.VMEM_SHARED((NPAD, D), jnp.float32),
        pltpu.VMEM((NCHUNKS, C), jnp.int32),
        pltpu.VMEM((NCHUNKS, C), jnp.int32),
        pltpu.VMEM((C, D), jnp.float32),
        pltpu.SemaphoreType.DMA,
    ],
)
def _scat_sc(g_hbm, srcs_hbm, dsts_hbm, zeros_hbm, out_hbm,
             acc, src_v, dst_v, rows, sem):
  cid = lax.axis_index("c")
  sid = lax.axis_index("s")
  wid = sid * NC + cid
  base = sid * ROWS_PT
  pltpu.sync_copy(zeros_hbm.at[pl.ds(base, ROWS_PT)],
                  acc.at[pl.ds(base, ROWS_PT)])
  pltpu.sync_copy(srcs_hbm.at[wid], src_v)
  pltpu.sync_copy(dsts_hbm.at[wid], dst_v)
  plsc.subcore_barrier()

  def body(j, carry):
    pltpu.async_copy(g_hbm.at[src_v.at[j]], rows, sem).wait()
    pltpu.sync_copy(rows, acc.at[dst_v.at[j]], add=True)
    return carry

  lax.fori_loop(0, NCHUNKS, body, None)
  plsc.subcore_barrier()
  pltpu.sync_copy(acc.at[pl.ds(base, ROWS_PT)],
                  out_hbm.at[cid, pl.ds(base, ROWS_PT)])


def _tc1_body(dref, xref, wref, gref, dvref):
  deg = dref[0, :, 0:1] + dref[1, :, 0:1] + 1.0
  dv = lax.rsqrt(deg)
  dvb = jnp.broadcast_to(dv, (TCB, D))
  gref[...] = dvb * jnp.dot(xref[...], wref[...],
                            preferred_element_type=jnp.float32)
  dvref[...] = dvb


_tc1 = pl.pallas_call(
    _tc1_body,
    grid=(NPAD // TCB,),
    in_specs=[
        pl.BlockSpec((2, TCB, D), lambda i: (0, i, 0)),
        pl.BlockSpec((TCB, D), lambda i: (i, 0)),
        pl.BlockSpec((D, D), lambda i: (0, 0)),
    ],
    out_specs=[
        pl.BlockSpec((TCB, D), lambda i: (i, 0)),
        pl.BlockSpec((TCB, D), lambda i: (i, 0)),
    ],
    out_shape=[
        jax.ShapeDtypeStruct((NPAD, D), jnp.float32),
        jax.ShapeDtypeStruct((NPAD, D), jnp.float32),
    ],
)


def _tc2_body(pref, gref, dvref, bref, wref, oref):
  t = dvref[...] * (pref[0] + pref[1] + gref[...]) + bref[...]
  h = jnp.maximum(t, 0.0)
  oref[...] = dvref[...] * jnp.dot(h, wref[...],
                                   preferred_element_type=jnp.float32)


_tc2 = pl.pallas_call(
    _tc2_body,
    grid=(NPAD // TCB,),
    in_specs=[
        pl.BlockSpec((2, TCB, D), lambda i: (0, i, 0)),
        pl.BlockSpec((TCB, D), lambda i: (i, 0)),
        pl.BlockSpec((TCB, D), lambda i: (i, 0)),
        pl.BlockSpec((1, D), lambda i: (0, 0)),
        pl.BlockSpec((D, D), lambda i: (0, 0)),
    ],
    out_specs=pl.BlockSpec((TCB, D), lambda i: (i, 0)),
    out_shape=jax.ShapeDtypeStruct((NPAD, D), jnp.float32),
)


def _tc3_body(pref, gref, dvref, bref, oref):
  oref[...] = dvref[...] * (pref[0] + pref[1] + gref[...]) + bref[...]


_tc3 = pl.pallas_call(
    _tc3_body,
    grid=(NPAD // TCB,),
    in_specs=[
        pl.BlockSpec((2, TCB, D), lambda i: (0, i, 0)),
        pl.BlockSpec((TCB, D), lambda i: (i, 0)),
        pl.BlockSpec((TCB, D), lambda i: (i, 0)),
        pl.BlockSpec((1, D), lambda i: (0, 0)),
    ],
    out_specs=pl.BlockSpec((TCB, D), lambda i: (i, 0)),
    out_shape=jax.ShapeDtypeStruct((NPAD, D), jnp.float32),
)


def kernel(x, edge_index, W1, b1, W2, b2):
  src = edge_index[0].astype(jnp.int32)
  dst = edge_index[1].astype(jnp.int32)
  e = src.shape[0]
  npadrows = 16
  padidx = N + (jnp.arange(EPAD - e, dtype=jnp.int32) % npadrows)
  srcs = jnp.concatenate([src, padidx]).reshape(NW, NCHUNKS, C)
  dsts = jnp.concatenate([dst, padidx]).reshape(NW, NCHUNKS, C)
  xp = jnp.pad(x, ((0, NPAD - N), (0, 0)))
  zeros_d = jnp.zeros((NPAD, D), jnp.float32)
  ones_d = jnp.ones((C, D), jnp.float32)

  degp = _deg_sc(dsts, ones_d, zeros_d)
  g1, dv = _tc1(degp, xp, W1)
  p = _scat_sc(g1, srcs, dsts, zeros_d)
  g2 = _tc2(p, g1, dv, b1.reshape(1, D), W2)
  q = _scat_sc(g2, srcs, dsts, zeros_d)
  out = _tc3(q, g2, dv, b2.reshape(1, D))
  return out[:N]


# speedup vs baseline: 27.1754x; 1.3781x over previous
import functools

import jax
import jax.numpy as jnp
from jax import lax
from jax.experimental import pallas as pl
from jax.experimental.pallas import tpu as pltpu
from jax.experimental.pallas import tpu_sc as plsc

N = 10000
D = 128
NPAD = 10240
NC = 2
NS = 16
NW = NC * NS
C = 128
NCHUNKS = 80
EPW = NCHUNKS * C
EPAD = NW * EPW
ROWS_PT = NPAD // NS
TCB = 1024
PACK_BITS = 14
PACK_MASK = (1 << PACK_BITS) - 1


def _sc_mesh():
  return plsc.VectorSubcoreMesh(core_axis_name="c", subcore_axis_name="s")


def _unpack_chunk(packed_v, j, src_c, dst_c):
  for k in range(C // 16):
    v = packed_v[j, pl.ds(k * 16, 16)]
    src_c[pl.ds(k * 16, 16)] = v & PACK_MASK
    dst_c[pl.ds(k * 16, 16)] = lax.shift_right_logical(v, PACK_BITS)


@functools.partial(
    pl.kernel,
    mesh=_sc_mesh(),
    out_type=jax.ShapeDtypeStruct((NC, NPAD, D), jnp.float32),
    scratch_types=[
        pltpu.VMEM_SHARED((NPAD, D), jnp.float32),
        pltpu.VMEM((NCHUNKS, C), jnp.int32),
        pltpu.VMEM((C,), jnp.int32),
        pltpu.VMEM((C,), jnp.int32),
        pltpu.VMEM((C,), jnp.int32),
        pltpu.VMEM((C,), jnp.int32),
        pltpu.VMEM((C,), jnp.int32),
        pltpu.VMEM((C, D), jnp.float32),
        pltpu.SemaphoreType.DMA,
        pltpu.SemaphoreType.DMA,
        pltpu.SemaphoreType.DMA,
        pltpu.SemaphoreType.DMA,
        pltpu.SemaphoreType.DMA,
    ],
)
def _deg_sc(edges_hbm, ones_hbm, zeros_hbm, out_hbm,
            acc, packed_v, junk_c, dst_c0, dst_c1, dst_c2, dst_c3, ones_v,
            semA, semB, semC, semD, semL):
  cid = lax.axis_index("c")
  sid = lax.axis_index("s")
  wid = sid * NC + cid
  base = sid * ROWS_PT
  pltpu.async_copy(zeros_hbm.at[pl.ds(base, ROWS_PT)],
                   acc.at[pl.ds(base, ROWS_PT)], semL)
  pltpu.sync_copy(edges_hbm.at[wid], packed_v)
  pltpu.sync_copy(ones_hbm, ones_v)
  pltpu.make_async_copy(zeros_hbm.at[pl.ds(base, ROWS_PT)],
                        acc.at[pl.ds(base, ROWS_PT)], semL).wait()
  plsc.subcore_barrier()

  dst_c = (dst_c0, dst_c1, dst_c2, dst_c3)
  sems = (semA, semB, semC, semD)
  for b in range(4):
    _unpack_chunk(packed_v, b, junk_c, dst_c[b])
    pltpu.async_copy(ones_v, acc.at[dst_c[b]], sems[b], add=True)

  def body(i, carry):
    j = 4 * i
    for b in range(4):
      pltpu.make_async_copy(ones_v, acc.at[dst_c[b]], sems[b]).wait()

      @pl.when(j + 4 + b < NCHUNKS)
      def _():
        _unpack_chunk(packed_v, j + 4 + b, junk_c, dst_c[b])
        pltpu.async_copy(ones_v, acc.at[dst_c[b]], sems[b], add=True)

    return carry

  lax.fori_loop(0, NCHUNKS // 4, body, None)
  plsc.subcore_barrier()
  pltpu.sync_copy(acc.at[pl.ds(base, ROWS_PT)],
                  out_hbm.at[cid, pl.ds(base, ROWS_PT)])


@functools.partial(
    pl.kernel,
    mesh=_sc_mesh(),
    out_type=jax.ShapeDtypeStruct((NC, NPAD, D), jnp.float32),
    scratch_types=[
        pltpu.VMEM_SHARED((NPAD, D), jnp.float32),
        pltpu.VMEM((NCHUNKS, C), jnp.int32),
        pltpu.VMEM((C,), jnp.int32),
        pltpu.VMEM((C,), jnp.int32),
        pltpu.VMEM((C,), jnp.int32),
        pltpu.VMEM((C,), jnp.int32),
        pltpu.VMEM((C, D), jnp.float32),
        pltpu.VMEM((C, D), jnp.float32),
        pltpu.SemaphoreType.DMA,
        pltpu.SemaphoreType.DMA,
    ],
)
def _scat_sc(g_hbm, edges_hbm, zeros_hbm, out_hbm,
             acc, packed_v, src_c0, src_c1, dst_c0, dst_c1,
             rows0, rows1, sem0, sem1):
  cid = lax.axis_index("c")
  sid = lax.axis_index("s")
  wid = sid * NC + cid
  base = sid * ROWS_PT

  @pl.when(cid == 0)
  def _():
    pltpu.async_copy(g_hbm.at[pl.ds(base, ROWS_PT)],
                     acc.at[pl.ds(base, ROWS_PT)], sem0)

  @pl.when(cid == 1)
  def _():
    pltpu.async_copy(zeros_hbm.at[pl.ds(base, ROWS_PT)],
                     acc.at[pl.ds(base, ROWS_PT)], sem0)

  pltpu.sync_copy(edges_hbm.at[wid], packed_v)

  @pl.when(cid == 0)
  def _():
    pltpu.make_async_copy(g_hbm.at[pl.ds(base, ROWS_PT)],
                          acc.at[pl.ds(base, ROWS_PT)], sem0).wait()

  @pl.when(cid == 1)
  def _():
    pltpu.make_async_copy(zeros_hbm.at[pl.ds(base, ROWS_PT)],
                          acc.at[pl.ds(base, ROWS_PT)], sem0).wait()

  plsc.subcore_barrier()

  _unpack_chunk(packed_v, 0, src_c0, dst_c0)
  pltpu.async_copy(g_hbm.at[src_c0], rows0, sem0)
  _unpack_chunk(packed_v, 1, src_c1, dst_c1)

  def body(i, carry):
    j = 2 * i
    pltpu.async_copy(g_hbm.at[src_c1], rows1, sem1)
    pltpu.make_async_copy(g_hbm.at[src_c0], rows0, sem0).wait()
    pltpu.sync_copy(rows0, acc.at[dst_c0], add=True)

    @pl.when(j + 2 < NCHUNKS)
    def _():
      _unpack_chunk(packed_v, j + 2, src_c0, dst_c0)
      pltpu.async_copy(g_hbm.at[src_c0], rows0, sem0)

    pltpu.make_async_copy(g_hbm.at[src_c1], rows1, sem1).wait()
    pltpu.sync_copy(rows1, acc.at[dst_c1], add=True)

    @pl.when(j + 3 < NCHUNKS)
    def _():
      _unpack_chunk(packed_v, j + 3, src_c1, dst_c1)

    return carry

  lax.fori_loop(0, NCHUNKS // 2, body, None)
  plsc.subcore_barrier()
  pltpu.sync_copy(acc.at[pl.ds(base, ROWS_PT)],
                  out_hbm.at[cid, pl.ds(base, ROWS_PT)])


def _tc1_body(dref, xref, wref, gref, dvref):
  deg = dref[0, :, 0:1] + dref[1, :, 0:1] + 1.0
  dv = lax.rsqrt(deg)
  dvb = jnp.broadcast_to(dv, (TCB, D))
  gref[...] = dvb * jnp.dot(xref[...], wref[...],
                            preferred_element_type=jnp.float32)
  dvref[...] = dv


_tc1 = pl.pallas_call(
    _tc1_body,
    grid=(NPAD // TCB,),
    in_specs=[
        pl.BlockSpec((2, TCB, D), lambda i: (0, i, 0)),
        pl.BlockSpec((TCB, D), lambda i: (i, 0)),
        pl.BlockSpec((D, D), lambda i: (0, 0)),
    ],
    out_specs=[
        pl.BlockSpec((TCB, D), lambda i: (i, 0)),
        pl.BlockSpec((TCB, 1), lambda i: (i, 0)),
    ],
    out_shape=[
        jax.ShapeDtypeStruct((NPAD, D), jnp.float32),
        jax.ShapeDtypeStruct((NPAD, 1), jnp.float32),
    ],
)


def _tc2_body(pref, dvref, bref, wref, oref):
  dvb = jnp.broadcast_to(dvref[...], (TCB, D))
  t = dvb * (pref[0] + pref[1]) + bref[...]
  h = jnp.maximum(t, 0.0)
  oref[...] = dvb * jnp.dot(h, wref[...],
                            preferred_element_type=jnp.float32)


_tc2 = pl.pallas_call(
    _tc2_body,
    grid=(NPAD // TCB,),
    in_specs=[
        pl.BlockSpec((2, TCB, D), lambda i: (0, i, 0)),
        pl.BlockSpec((TCB, 1), lambda i: (i, 0)),
        pl.BlockSpec((1, D), lambda i: (0, 0)),
        pl.BlockSpec((D, D), lambda i: (0, 0)),
    ],
    out_specs=pl.BlockSpec((TCB, D), lambda i: (i, 0)),
    out_shape=jax.ShapeDtypeStruct((NPAD, D), jnp.float32),
)


def _tc3_body(pref, dvref, bref, oref):
  dvb = jnp.broadcast_to(dvref[...], (TCB, D))
  oref[...] = dvb * (pref[0] + pref[1]) + bref[...]


_tc3 = pl.pallas_call(
    _tc3_body,
    grid=(NPAD // TCB,),
    in_specs=[
        pl.BlockSpec((2, TCB, D), lambda i: (0, i, 0)),
        pl.BlockSpec((TCB, 1), lambda i: (i, 0)),
        pl.BlockSpec((1, D), lambda i: (0, 0)),
    ],
    out_specs=pl.BlockSpec((TCB, D), lambda i: (i, 0)),
    out_shape=jax.ShapeDtypeStruct((NPAD, D), jnp.float32),
)


def kernel(x, edge_index, W1, b1, W2, b2):
  src = edge_index[0].astype(jnp.int32)
  dst = edge_index[1].astype(jnp.int32)
  e = src.shape[0]
  npadrows = 16
  padidx = N + (jnp.arange(EPAD - e, dtype=jnp.int32) % npadrows)
  srcp = jnp.concatenate([src, padidx])
  dstp = jnp.concatenate([dst, padidx])
  edges = (srcp | (dstp << PACK_BITS)).reshape(NW, NCHUNKS, C)
  xp = jnp.pad(x, ((0, NPAD - N), (0, 0)))
  zeros_d = jnp.zeros((NPAD, D), jnp.float32)
  ones_d = jnp.ones((C, D), jnp.float32)

  degp = _deg_sc(edges, ones_d, zeros_d)
  g1, dv = _tc1(degp, xp, W1)
  p = _scat_sc(g1, edges, zeros_d)
  g2 = _tc2(p, dv, b1.reshape(1, D), W2)
  q = _scat_sc(g2, edges, zeros_d)
  out = _tc3(q, dv, b2.reshape(1, D))
  return out[:N]
